# baseline (speedup 1.0000x reference)
import functools

import jax
import jax.numpy as jnp
from jax import lax
from jax.experimental import pallas as pl
from jax.experimental.pallas import tpu as pltpu
from jax.experimental.pallas import tpu_sc as plsc

_CH = 80000
_ACC = _CH + 16


@functools.lru_cache(maxsize=None)
def _make_sc_scatter(KP, TR, OR, C):
    NT = 16
    PT = KP // NT
    NB = PT // 128
    mesh = plsc.VectorSubcoreMesh(core_axis_name="c", subcore_axis_name="s")
    ZR = _ACC // NT
    DR = _CH // NT

    @functools.partial(
        pl.kernel, mesh=mesh,
        out_type=jax.ShapeDtypeStruct((8, OR, 16), jnp.float32),
        compiler_params=pltpu.CompilerParams(use_tc_tiling_on_sc=False),
        scratch_types=[
            pltpu.VMEM_SHARED((_ACC, 16), jnp.float32),
            pltpu.VMEM((PT,), jnp.int32),
            pltpu.VMEM((PT,), jnp.int32),
            pltpu.VMEM((256, 16), jnp.float32),
            pltpu.VMEM((1, 128), jnp.int32),
            pltpu.VMEM((1, 128), jnp.int32),
            pltpu.VMEM((128, 16), jnp.float32),
            pltpu.SemaphoreType.DMA,
        ])
    def k(table_h, gidx_h, gdst_h, zrows_h, out_h,
          acc, gstage, dstage, zbuf, gb1, db1, rowbuf, sem):
        core = lax.axis_index("c")
        tid = lax.axis_index("s")
        pltpu.sync_copy(zrows_h, zbuf)
        e0 = tid * PT
        pltpu.sync_copy(gidx_h.at[pl.ds(e0, PT)], gstage)
        pltpu.sync_copy(gdst_h.at[pl.ds(e0, PT)], dstage)
        for ci in range(C // 2):
            chunk = core * (C // 2) + ci
            base = chunk * _CH
            for s in range(8):
                z0 = tid * ZR
                nfull = ZR // 256
                for q in range(nfull):
                    pltpu.sync_copy(zbuf, acc.at[pl.ds(z0 + q * 256, 256)])
                rem = ZR - nfull * 256
                if rem:
                    pltpu.sync_copy(zbuf.at[pl.ds(0, rem)],
                                    acc.at[pl.ds(z0 + nfull * 256, rem)])
                plsc.subcore_barrier()

                def body(j, _):
                    for c in range(8):
                        off = j * 128 + c * 16
                        gv = gstage[pl.ds(off, 16)] + s
                        dv = dstage[pl.ds(off, 16)] - base
                        ok = (dv >= 0) & (dv < _CH)
                        dv = jnp.where(ok, dv, _CH)
                        gb1[0, pl.ds(c * 16, 16)] = gv
                        db1[0, pl.ds(c * 16, 16)] = dv
                    pltpu.async_copy(table_h.at[gb1.at[0]], rowbuf, sem).wait()
                    pltpu.sync_copy(rowbuf, acc.at[db1.at[0]], add=True)
                    return _

                lax.fori_loop(0, NB, body, None)
                plsc.subcore_barrier()
                d0 = tid * DR
                pltpu.sync_copy(acc.at[pl.ds(d0, DR)],
                                out_h.at[s, pl.ds(base + d0, DR)])
                plsc.subcore_barrier()

    return k


def _sc_scatter_rows(table2d, gidx8, gdst, OR, C):
    K = gidx8.shape[0]
    KP = -(-K // 2048) * 2048
    if KP != K:
        gidx8 = jnp.pad(gidx8, (0, KP - K))
        gdst = jnp.pad(gdst, (0, KP - K), constant_values=-1)
    zrows = jnp.zeros((256, 16), jnp.float32)
    k = _make_sc_scatter(KP, table2d.shape[0], OR, C)
    return k(table2d, gidx8, gdst, zrows)


def _seg_sum(vals, seg, num):
    return jax.ops.segment_sum(vals, seg, num_segments=num)


def _combine_body(a_ref, b_ref, o_ref):
    o_ref[...] = a_ref[...] + b_ref[...]


def _pl_add(a, b):
    E, F = a.shape
    blk = 2000
    return pl.pallas_call(
        _combine_body,
        out_shape=jax.ShapeDtypeStruct((E, F), jnp.float32),
        grid=(E // blk,),
        in_specs=[pl.BlockSpec((blk, F), lambda i: (i, 0)),
                  pl.BlockSpec((blk, F), lambda i: (i, 0))],
        out_specs=pl.BlockSpec((blk, F), lambda i: (i, 0)),
    )(a, b)


def kernel(x, edge_index, edge_attr, line_graph_edge_index, edge_index_batch, params):
    p = params
    N, F = x.shape
    E = edge_index.shape[1]
    B = 128
    T = 3
    lg0 = line_graph_edge_index[0]
    lg1 = line_graph_edge_index[1]
    ei0, ei1 = edge_index[0], edge_index[1]
    batch = edge_index_batch

    edge_u = x @ p['Wu']
    edge_v = x @ p['Wv']
    edge_uv = edge_attr @ p['We']
    ea = (edge_u[ei0] + edge_v[ei1] + edge_uv) / 3.0

    indeg = _seg_sum(jnp.ones((lg1.shape[0],), jnp.float32), lg1, E)
    dis = (indeg + 1.0) ** -0.5

    vs2 = p['gat_W'] @ p['gat_att_src']
    vd2 = p['gat_W'] @ p['gat_att_dst']

    lg0x8 = lg0 * 8

    out = ea
    out_list = []
    gout_list = []
    for _ in range(T):
        aggs = _sc_scatter_rows(out.reshape(E * 8, 16), lg0x8, lg1, E, 2)
        agg = aggs.transpose(1, 0, 2).reshape(E, F)
        out = _pl_add(ea, agg)

        h = out @ p['gat_W']
        a_s = out @ vs2
        a_d = out @ vd2
        h1 = (out @ p['att_gcn_W'])[:, 0]
        score_f = out @ p['fbtl_W'] + p['fbtl_b']

        gh1 = dis * h1
        s_lg = _seg_sum(gh1[lg0], lg1, E)
        score_s = dis * s_lg + dis * dis * h1 + p['att_gcn_b'][0]
        score = score_s[:, None] * 0.6 + score_f * 0.4

        u = a_s[lg0] + a_d[lg1]
        pos = u >= 0.0
        cval = jnp.where(pos, jnp.exp(a_s[lg0]), jnp.exp(0.2 * a_s[lg0]))
        cpos = _seg_sum(jnp.where(pos, cval, 0.0), lg1, E)
        cneg = _seg_sum(jnp.where(pos, 0.0, cval), lg1, E)
        hA = jnp.exp(a_s)[:, None] * h
        hB = jnp.exp(0.2 * a_s)[:, None] * h
        rows = jnp.where(pos[:, None], hA[lg0], hB[lg0])
        Spos = _seg_sum(jnp.where(pos[:, None], rows, 0.0), lg1, E)
        Sneg = _seg_sum(jnp.where(pos[:, None], 0.0, rows), lg1, E)
        e_self = jnp.exp(jax.nn.leaky_relu(a_s + a_d, 0.2))
        ead = jnp.exp(a_d)
        ead2 = jnp.exp(0.2 * a_d)
        z = ead * cpos + ead2 * cneg + e_self
        num = ead[:, None] * Spos + ead2[:, None] * Sneg + e_self[:, None] * h
        xf = num / (z + 1e-16)[:, None] + p['gat_b']

        es = jnp.exp(score)
        zb = _seg_sum(es, batch, B)
        scores = es / (zb[batch] + 1e-16)
        gout = _seg_sum(xf * scores, batch, B)

        out_list.append(out)
        gout_list.append(jnp.tanh(gout @ p['lin_gout_W'] + p['lin_gout_b']))

    gout_all = jnp.stack(gout_list, axis=-1)
    out_all = jnp.stack(out_list, axis=-1)
    ws = (gout_all * p['a']).sum(1, keepdims=True) + p['a_bias']
    ws = jax.nn.softmax(ws, axis=-1)
    we = ws[batch, 0, :]
    o = (out_all * we[:, None, :]).sum(-1)
    x2 = x + _seg_sum(o, ei1, N)

    def bn(v, g, b):
        return g * (v - v.mean(0)) / jnp.sqrt(v.var(0) + 1e-5) + b

    def prelu(v, w):
        return jnp.where(v >= 0.0, v, w * v)

    y = bn(x2, p['bn1_g'], p['bn1_b']) @ p['l1_W'] + p['l1_b']
    hh = prelu(bn(y, p['bn2_g'], p['bn2_b']), p['pr2']) @ p['l2_W'] + p['l2_b']
    hh = prelu(bn(hh, p['bn3_g'], p['bn3_b']), p['pr3']) @ p['l3_W'] + p['l3_b']
    y = (hh + y) / 2.0
    hh = prelu(bn(y, p['bn4_g'], p['bn4_b']), p['pr4']) @ p['l4_W'] + p['l4_b']
    y = (hh + y) / 2.0
    y = prelu(bn(y, p['bn5_g'], p['bn5_b']), p['pr5']) @ p['l5_W'] + p['l5_b']
    return y
